# Initial kernel scaffold; baseline (speedup 1.0000x reference)
import jax
import jax.numpy as jnp
from jax.experimental import pallas as pl


def kernel(input, embedding):
    raise NotImplementedError("write your pallas kernel here")


# speedup vs baseline: 1.0638x; 1.0638x over previous
import functools

import jax
import jax.numpy as jnp
from jax import lax
from jax.experimental import pallas as pl
from jax.experimental.pallas import tpu as pltpu
from jax.experimental.pallas import tpu_sc as plsc

_E = 32
_K = 8192
_BM = 256


def _dist_argmin_block(f_ref, rn_ref, e_ref, code_ref, dist_ref):
    f = f_ref[...]
    e = e_ref[...]
    fe = jax.lax.dot_general(
        f, e, (((1,), (0,)), ((), ())),
        preferred_element_type=jnp.float32)
    coln = jnp.sum(e * e, axis=0, keepdims=True)
    dist = (rn_ref[...] - 2.0 * fe) + coln
    dist_ref[...] = dist
    dist = dist_ref[...]
    minval = jnp.min(dist, axis=1, keepdims=True)
    iota = jax.lax.broadcasted_iota(jnp.int32, (_BM, _K), 1)
    cand = jnp.where(dist == minval, iota, _K)
    code_ref[0, 0, :] = jnp.min(cand, axis=1).astype(jnp.int32)


def _loss_block(q_ref, f_ref, dsum_ref):
    d = q_ref[...] - f_ref[...]
    dsum_ref[...] = jnp.sum(d * d, axis=(0, 1), keepdims=True)


def _sc_gather(table, idx, b_per_w, num_cores):
    mesh = plsc.VectorSubcoreMesh(core_axis_name="c", subcore_axis_name="s")

    @functools.partial(
        pl.kernel, mesh=mesh,
        compiler_params=pltpu.CompilerParams(use_tc_tiling_on_sc=False),
        out_type=jax.ShapeDtypeStruct(idx.shape + (table.shape[1],),
                                      jnp.float32),
        scratch_types=[
            pltpu.VMEM((b_per_w,), jnp.int32),
            pltpu.VMEM((b_per_w, table.shape[1]), jnp.float32),
            pltpu.SemaphoreType.DMA,
        ],
    )
    def gather(table_hbm, idx_hbm, out_hbm, idx_v, rows_v, sem):
        wid = lax.axis_index("s") * num_cores + lax.axis_index("c")
        base = wid * b_per_w
        pltpu.sync_copy(idx_hbm.at[pl.ds(base, b_per_w)], idx_v)
        pltpu.async_copy(table_hbm.at[idx_v], rows_v, sem).wait()
        pltpu.sync_copy(rows_v, out_hbm.at[pl.ds(base, b_per_w)])

    return gather(table, idx)


def kernel(input, embedding):
    m = input.shape[0] * input.shape[1]
    flat = input.reshape(m, _E)
    rown = jnp.sum(flat ** 2, axis=1, keepdims=True)
    grid = m // _BM
    code3 = pl.pallas_call(
        _dist_argmin_block,
        grid=(grid,),
        in_specs=[
            pl.BlockSpec((_BM, _E), lambda i: (i, 0)),
            pl.BlockSpec((_BM, 1), lambda i: (i, 0)),
            pl.BlockSpec((_E, _K), lambda i: (0, 0)),
        ],
        out_specs=pl.BlockSpec((1, 1, _BM), lambda i: (i, 0, 0)),
        out_shape=jax.ShapeDtypeStruct((grid, 1, _BM), jnp.int32),
        scratch_shapes=[pltpu.VMEM((_BM, _K), jnp.float32)],
    )(flat, rown, embedding)
    code_flat = code3.reshape(m)

    info = plsc.get_sparse_core_info()
    nw = info.num_cores * info.num_subcores
    q = _sc_gather(embedding.T, code_flat, m // nw, info.num_cores)

    dsum = pl.pallas_call(
        _loss_block,
        out_shape=jax.ShapeDtypeStruct((1, 1), jnp.float32),
    )(q, flat)

    quantize = q.reshape(input.shape)
    code = code3.reshape(input.shape[:-1])
    diff = 0.25 * (dsum[0, 0] / (m * _E))
    return quantize, diff, code
